# baseline (speedup 1.0000x reference)
import jax
import jax.numpy as jnp
from jax import lax
from jax.experimental import pallas as pl
from jax.experimental.pallas import tpu as pltpu
from jax.experimental.pallas import tpu_sc as plsc

_N = 10000
_E = 320000
_D = 128

_NC = 2
_NS = 16
_NW = _NC * _NS
_CHUNK = 128
_BCH = 16
_B0 = 10
_B1 = 0
_CHMAX = _BCH * max(_B0, _B1)
_NCH = _NS * _BCH * (_B0 + _B1)
_EPAD = _NCH * _CHUNK
_ACC = 10240
_STRIPE = _ACC // _NS
_NPAD = 10016

_MESH = plsc.VectorSubcoreMesh(
    core_axis_name="c", subcore_axis_name="s", num_cores=_NC, num_subcores=_NS)
_SC_PARAMS = pltpu.CompilerParams(needs_layout_passes=False)


def _fill16(ref, val):
  v = jnp.full((16,), val, jnp.float32)
  for g in range(8):
    ref[pl.ds(16 * g, 16)] = v


def _zero_rows(ref):
  z = jnp.zeros((16,), jnp.float32)

  def body(r, _):
    for g in range(8):
      ref[r, pl.ds(16 * g, 16)] = z
    return 0

  lax.fori_loop(0, ref.shape[0], body, 0)


def _deg_body(dst_hbm, degp_hbm, dst_sm, ones_v, zrow_v, acc_sh):
  cid = lax.axis_index("c")
  sid = lax.axis_index("s")
  wid = cid * _NS + sid

  _fill16(ones_v, 1.0)
  _fill16(zrow_v, 0.0)
  for k in range(_STRIPE // 128):
    pltpu.sync_copy(zrow_v, acc_sh.at[pl.ds(sid * _STRIPE + 128 * k, 128)])
  plsc.subcore_barrier()

  def blk(b, _):
    pltpu.sync_copy(dst_hbm.at[wid, pl.ds(b * _BCH, _BCH)], dst_sm)

    def body(j, _):
      pltpu.sync_copy(ones_v, acc_sh.at[dst_sm.at[j]], add=True)
      return 0

    lax.fori_loop(0, _BCH, body, 0)
    return 0

  lax.fori_loop(0, jnp.where(cid == 0, _B0, _B1), blk, 0)
  plsc.subcore_barrier()
  for k in range(_STRIPE // 128):
    pltpu.sync_copy(acc_sh.at[pl.ds(sid * _STRIPE + 128 * k, 128)], zrow_v)
    pltpu.sync_copy(zrow_v, degp_hbm.at[cid, pl.ds(sid * _STRIPE + 128 * k, 128)])


def _deg_kernel(dst3):
  return pl.kernel(
      _deg_body,
      out_type=jax.ShapeDtypeStruct((_NC, _ACC), jnp.float32),
      mesh=_MESH,
      compiler_params=_SC_PARAMS,
      scratch_types=[
          pltpu.VMEM((_BCH, _CHUNK), jnp.int32),
          pltpu.VMEM((_CHUNK,), jnp.float32),
          pltpu.VMEM((_CHUNK,), jnp.float32),
          pltpu.VMEM_SHARED((_ACC,), jnp.float32),
      ],
  )(dst3)


def _nblk(cid):
  return jnp.where(cid == 0, _B0, _B1)


def _gcn_body(y_hbm, src_hbm, dst_hbm, zp_hbm,
              src_sm, dst_sm, rows0, rows1, sem0, sem1, acc_sh):
  cid = lax.axis_index("c")
  sid = lax.axis_index("s")
  wid = cid * _NS + sid

  _zero_rows(rows0)
  for k in range(_STRIPE // 128):
    pltpu.sync_copy(rows0, acc_sh.at[pl.ds(sid * _STRIPE + 128 * k, 128)])
  plsc.subcore_barrier()

  def blk(b, _):
    pltpu.sync_copy(src_hbm.at[wid, pl.ds(b * _BCH, _BCH)], src_sm)
    pltpu.sync_copy(dst_hbm.at[wid, pl.ds(b * _BCH, _BCH)], dst_sm)
    pltpu.async_copy(y_hbm.at[src_sm.at[0]], rows0, sem0)
    pltpu.async_copy(y_hbm.at[src_sm.at[1]], rows1, sem1)

    def pair(i, _):
      j0 = 2 * i
      pltpu.make_async_copy(y_hbm.at[src_sm.at[j0]], rows0, sem0).wait()
      pltpu.sync_copy(rows0, acc_sh.at[dst_sm.at[j0]], add=True)

      @pl.when(j0 + 2 < _BCH)
      def _():
        pltpu.async_copy(y_hbm.at[src_sm.at[j0 + 2]], rows0, sem0)

      pltpu.make_async_copy(y_hbm.at[src_sm.at[j0 + 1]], rows1, sem1).wait()
      pltpu.sync_copy(rows1, acc_sh.at[dst_sm.at[j0 + 1]], add=True)

      @pl.when(j0 + 3 < _BCH)
      def _():
        pltpu.async_copy(y_hbm.at[src_sm.at[j0 + 3]], rows1, sem1)

      return 0

    lax.fori_loop(0, _BCH // 2, pair, 0)
    return 0

  lax.fori_loop(0, _nblk(cid), blk, 0)
  plsc.subcore_barrier()
  for k in range(_STRIPE // 128):
    pltpu.sync_copy(acc_sh.at[pl.ds(sid * _STRIPE + 128 * k, 128)], rows0)
    pltpu.sync_copy(rows0, zp_hbm.at[cid, pl.ds(sid * _STRIPE + 128 * k, 128)])


def _gcn_kernel(y, src3, dst3):
  return pl.kernel(
      _gcn_body,
      out_type=jax.ShapeDtypeStruct((_NC, _ACC, _D), jnp.float32),
      mesh=_MESH,
      compiler_params=_SC_PARAMS,
      scratch_types=[
          pltpu.VMEM((_BCH, _CHUNK), jnp.int32),
          pltpu.VMEM((_BCH, _CHUNK), jnp.int32),
          pltpu.VMEM((_CHUNK, _D), jnp.float32),
          pltpu.VMEM((_CHUNK, _D), jnp.float32),
          pltpu.SemaphoreType.DMA,
          pltpu.SemaphoreType.DMA,
          pltpu.VMEM_SHARED((_ACC, _D), jnp.float32),
      ],
  )(y, src3, dst3)


_GCHUNK = 64
_GBCH = 32
_GCHMAX = _GBCH * max(_B0, _B1)


def _gat_chunk(src_sm, dst_sm, gs, as_v, ad_v, w_v, j):
  for g in range(_GCHUNK // 16):
    si = src_sm[j, pl.ds(16 * g, 16)]
    di = dst_sm[j, pl.ds(16 * g, 16)]
    av = plsc.load_gather(as_v, [si])
    dv = plsc.load_gather(ad_v, [di])
    td = gs + dv
    cv = jnp.where(td > 0, td, 0.2 * td)
    t = av + dv
    t = jnp.where(t > 0, t, 0.2 * t)
    w_v[pl.ds(16 * g, 16)] = jnp.exp(t - cv)


def _gat_scale(rows_v, w_v):
  def sbody(e, _):
    we = plsc.load_gather(w_v, [jnp.full((16,), e, jnp.int32)])
    for g in range(8):
      rows_v[e, pl.ds(16 * g, 16)] = rows_v[e, pl.ds(16 * g, 16)] * we
    return 0

  lax.fori_loop(0, _GCHUNK, sbody, 0, unroll=2)


def _gat_body(xl_hbm, src_hbm, dst_hbm, gs_hbm, as_hbm, ad_hbm,
              nump_hbm, denp_hbm,
              src_sm, dst_sm, rows0, rows1, w0_v, w1_v, gs_v, as_v, ad_v,
              sem0, sem1, num_sh, den_sh):
  cid = lax.axis_index("c")
  sid = lax.axis_index("s")
  wid = cid * _NS + sid

  _zero_rows(rows0)
  for k in range(_STRIPE // _GCHUNK):
    pltpu.sync_copy(rows0, num_sh.at[pl.ds(sid * _STRIPE + _GCHUNK * k, _GCHUNK)])
  for k in range(_STRIPE // 128):
    pltpu.sync_copy(rows0.at[0], den_sh.at[pl.ds(sid * _STRIPE + 128 * k, 128)])
  plsc.subcore_barrier()

  pltpu.sync_copy(gs_hbm, gs_v)
  pltpu.sync_copy(as_hbm, as_v)
  pltpu.sync_copy(ad_hbm, ad_v)

  def blk(b, _):
    pltpu.sync_copy(src_hbm.at[wid, pl.ds(b * _GBCH, _GBCH)], src_sm)
    pltpu.sync_copy(dst_hbm.at[wid, pl.ds(b * _GBCH, _GBCH)], dst_sm)
    pltpu.async_copy(xl_hbm.at[src_sm.at[0]], rows0, sem0)
    pltpu.async_copy(xl_hbm.at[src_sm.at[1]], rows1, sem1)
    gs = gs_v[...]

    def pair(i, _):
      j0 = 2 * i
      _gat_chunk(src_sm, dst_sm, gs, as_v, ad_v, w0_v, j0)
      pltpu.make_async_copy(xl_hbm.at[src_sm.at[j0]], rows0, sem0).wait()
      _gat_scale(rows0, w0_v)
      pltpu.sync_copy(rows0, num_sh.at[dst_sm.at[j0]], add=True)
      pltpu.sync_copy(w0_v, den_sh.at[dst_sm.at[j0]], add=True)

      @pl.when(j0 + 2 < _GBCH)
      def _():
        pltpu.async_copy(xl_hbm.at[src_sm.at[j0 + 2]], rows0, sem0)

      _gat_chunk(src_sm, dst_sm, gs, as_v, ad_v, w1_v, j0 + 1)
      pltpu.make_async_copy(xl_hbm.at[src_sm.at[j0 + 1]], rows1, sem1).wait()
      _gat_scale(rows1, w1_v)
      pltpu.sync_copy(rows1, num_sh.at[dst_sm.at[j0 + 1]], add=True)
      pltpu.sync_copy(w1_v, den_sh.at[dst_sm.at[j0 + 1]], add=True)

      @pl.when(j0 + 3 < _GBCH)
      def _():
        pltpu.async_copy(xl_hbm.at[src_sm.at[j0 + 3]], rows1, sem1)

      return 0

    lax.fori_loop(0, _GBCH // 2, pair, 0)
    return 0

  lax.fori_loop(0, _nblk(cid), blk, 0)
  plsc.subcore_barrier()
  for k in range(_STRIPE // _GCHUNK):
    pltpu.sync_copy(num_sh.at[pl.ds(sid * _STRIPE + _GCHUNK * k, _GCHUNK)], rows0)
    pltpu.sync_copy(rows0, nump_hbm.at[cid, pl.ds(sid * _STRIPE + _GCHUNK * k, _GCHUNK)])
  for k in range(_STRIPE // 128):
    pltpu.sync_copy(den_sh.at[pl.ds(sid * _STRIPE + 128 * k, 128)], rows0.at[0])
    pltpu.sync_copy(rows0.at[0], denp_hbm.at[cid, pl.ds(sid * _STRIPE + 128 * k, 128)])


def _gat_kernel(xl, src3g, dst3g, gsl, asl, adl):
  return pl.kernel(
      _gat_body,
      out_type=(jax.ShapeDtypeStruct((_NC, _ACC, _D), jnp.float32),
                jax.ShapeDtypeStruct((_NC, _ACC), jnp.float32)),
      mesh=_MESH,
      compiler_params=_SC_PARAMS,
      scratch_types=[
          pltpu.VMEM((_GBCH, _GCHUNK), jnp.int32),
          pltpu.VMEM((_GBCH, _GCHUNK), jnp.int32),
          pltpu.VMEM((_GCHUNK, _D), jnp.float32),
          pltpu.VMEM((_GCHUNK, _D), jnp.float32),
          pltpu.VMEM((_GCHUNK,), jnp.float32),
          pltpu.VMEM((_GCHUNK,), jnp.float32),
          pltpu.VMEM((16,), jnp.float32),
          pltpu.VMEM((_NPAD,), jnp.float32),
          pltpu.VMEM((_NPAD,), jnp.float32),
          pltpu.SemaphoreType.DMA,
          pltpu.SemaphoreType.DMA,
          pltpu.VMEM_SHARED((_ACC, _D), jnp.float32),
          pltpu.VMEM_SHARED((_ACC,), jnp.float32),
      ],
  )(xl, src3g, dst3g, gsl, asl, adl)


def _dot(a, b):
  return jnp.dot(a, b, preferred_element_type=jnp.float32)


def _tc_a_body(ev, w1, b1, w2, b2, wg, degs, y_ref, dis_ref):
  h = jnp.maximum(_dot(ev[...], w1[...]) + b1[...], 0.0)
  emb = _dot(h, w2[...]) + b2[...]
  xt = _dot(emb, wg[...])
  deg = jnp.sum(degs[...], axis=1, keepdims=True) + 1.0
  dis = lax.rsqrt(deg)
  y_ref[...] = xt * dis
  dis_ref[...] = dis


def _tc_a(ev, w1, b1, w2, b2, wg, degs):
  return pl.pallas_call(
      _tc_a_body,
      out_shape=(jax.ShapeDtypeStruct((_N, _D), jnp.float32),
                 jax.ShapeDtypeStruct((_N, 1), jnp.float32)),
  )(ev, w1, b1, w2, b2, wg, degs)


def _tc_b_body(y, dis, z0, z1, wgat, asrc, adst, bgcn, xl_ref, sc_ref):
  zsum = z0[...] + z1[...] + y[...]
  gcn = jnp.maximum(dis[...] * zsum + bgcn[...], 0.0)
  xl = _dot(gcn, wgat[...])
  a_s = _dot(xl, asrc[...])
  a_d = _dot(xl, adst[...])
  gmax = jnp.max(a_s)
  td = gmax + a_d
  cc = jnp.where(td > 0, td, 0.2 * td)
  ts = a_s + a_d
  ts = jnp.where(ts > 0, ts, 0.2 * ts)
  wself = jnp.exp(ts - cc)
  gbc = jnp.full((_N, 1), 1.0, jnp.float32) * gmax
  xl_ref[...] = xl
  sc_ref[...] = jnp.concatenate([a_s, a_d, wself, gbc], axis=1)


def _tc_b(y, dis, z0, z1, wgat, asrc, adst, bgcn):
  return pl.pallas_call(
      _tc_b_body,
      out_shape=(jax.ShapeDtypeStruct((_N, _D), jnp.float32),
                 jax.ShapeDtypeStruct((_N, 4), jnp.float32)),
  )(y, dis, z0, z1, wgat, asrc, adst, bgcn)


_BLK = 2000


def _tc_c_body(h, xl, n0, n1, dens, sc, bgat, wg, bg, wr1, br1, wr2, br2,
               ws1, bs1, ws2, bs2, d_ref, hf_ref, p_ref):
  H = h[...]
  XL = xl[...]
  ws = sc[...][:, 2:3]
  den = jnp.sum(dens[...], axis=1, keepdims=True) + ws
  num = n0[...] + n1[...] + ws * XL
  diff = jnp.maximum(num / jnp.maximum(den, 1e-16) + bgat[...], 0.0)
  WG = wg[...]
  gate = jax.nn.sigmoid(_dot(H, WG[:128]) + _dot(diff, WG[128:]) + bg[...])
  WR1 = wr1[...]
  hr = jnp.maximum(_dot(H, WR1[:128]) + _dot(diff, WR1[128:]) + br1[...], 0.0)
  draw = _dot(hr, wr2[...]) + br2[...]
  delta = gate * draw
  hf = H + delta
  p = _dot(jnp.maximum(_dot(hf, ws1[...]) + bs1[...], 0.0), ws2[...]) + bs2[...]
  d_ref[...] = delta
  hf_ref[...] = hf
  p_ref[...] = p


def _tc_c(h, xl, n0, n1, dens, sc, bgat, wg, bg, wr1, br1, wr2, br2,
          ws1, bs1, ws2, bs2):
  nb = _N // _BLK

  def full(shape):
    return pl.BlockSpec(shape, lambda i: tuple(0 for _ in shape))

  return pl.pallas_call(
      _tc_c_body,
      grid=(nb,),
      in_specs=[
          pl.BlockSpec((_BLK, _D), lambda i: (i, 0)),
          pl.BlockSpec((_BLK, _D), lambda i: (i, 0)),
          pl.BlockSpec((_BLK, _D), lambda i: (i, 0)),
          pl.BlockSpec((_BLK, _D), lambda i: (i, 0)),
          pl.BlockSpec((_BLK, 2), lambda i: (i, 0)),
          pl.BlockSpec((_BLK, 4), lambda i: (i, 0)),
          full((1, _D)),
          full((2 * _D, _D)),
          full((1, _D)),
          full((2 * _D, _D)),
          full((1, _D)),
          full((_D, _D)),
          full((1, _D)),
          full((_D, 32)),
          full((1, 32)),
          full((32, 1)),
          full((1, 1)),
      ],
      out_specs=[
          pl.BlockSpec((_BLK, _D), lambda i: (i, 0)),
          pl.BlockSpec((_BLK, _D), lambda i: (i, 0)),
          pl.BlockSpec((_BLK, 1), lambda i: (i, 0)),
      ],
      out_shape=(jax.ShapeDtypeStruct((_N, _D), jnp.float32),
                 jax.ShapeDtypeStruct((_N, _D), jnp.float32),
                 jax.ShapeDtypeStruct((_N, 1), jnp.float32)),
  )(h, xl, n0, n1, dens, sc, bgat, wg, bg, wr1, br1, wr2, br2,
    ws1, bs1, ws2, bs2)


@jax.jit
def kernel(H_adapted_t, event_vector, edge_index,
           W_e1, b_e1, W_e2, b_e2,
           W_gcn, b_gcn,
           W_gat, a_src, a_dst, b_gat,
           W_gate, b_gate,
           W_r1, b_r1, W_r2, b_r2,
           W_s1, b_s1, W_s2, b_s2):
  src = edge_index[0]
  dst = edge_index[1]
  pad = _EPAD - _E
  i32 = jnp.int32
  src_p = jnp.concatenate([src, jnp.zeros((pad,), i32)])
  dst_p = jnp.concatenate([dst, jnp.full((pad,), _N, i32)])

  def skew(flat, nchunk, chunk, bch):
    f = flat.reshape(nchunk, chunk)
    n0 = _NS * bch * _B0
    chmax = bch * max(_B0, _B1)
    a = f[:n0].reshape(_NS, bch * _B0, chunk)
    a = jnp.pad(a, ((0, 0), (0, chmax - bch * _B0), (0, 0)))
    b = f[n0:].reshape(_NS, bch * _B1, chunk)
    b = jnp.pad(b, ((0, 0), (0, chmax - bch * _B1), (0, 0)))
    return jnp.concatenate([a, b], axis=0)

  src3 = skew(src_p, _NCH, _CHUNK, _BCH)
  dst3 = skew(dst_p, _NCH, _CHUNK, _BCH)

  degp = _deg_kernel(dst3)
  degs = jnp.stack([degp[0, :_N], degp[1, :_N]], axis=1)

  y, dis = _tc_a(event_vector, W_e1, b_e1.reshape(1, _D), W_e2,
                 b_e2.reshape(1, _D), W_gcn, degs)

  zp = _gcn_kernel(y, src3, dst3)

  xl, sc = _tc_b(y, dis, zp[0, :_N], zp[1, :_N], W_gat,
                 a_src.reshape(_D, 1), a_dst.reshape(_D, 1),
                 b_gcn.reshape(1, _D))

  zpadf = jnp.zeros((_NPAD - _N,), jnp.float32)
  asl = jnp.concatenate([sc[:, 0], zpadf])
  adl = jnp.concatenate([sc[:, 1], zpadf])
  gsl = sc[0:16, 3]

  src3g = skew(src_p, 2 * _NCH, _GCHUNK, _GBCH)
  dst3g = skew(dst_p, 2 * _NCH, _GCHUNK, _GBCH)
  nump, denp = _gat_kernel(xl, src3g, dst3g, gsl, asl, adl)
  dens = jnp.stack([denp[0, :_N], denp[1, :_N]], axis=1)

  delta, hf, p = _tc_c(H_adapted_t, xl, nump[0, :_N], nump[1, :_N], dens, sc,
                       b_gat.reshape(1, _D), W_gate, b_gate.reshape(1, _D),
                       W_r1, b_r1.reshape(1, _D), W_r2, b_r2.reshape(1, _D),
                       W_s1, b_s1.reshape(1, 32), W_s2, b_s2.reshape(1, 1))
  return delta, hf, p.reshape(_N)


# speedup vs baseline: 1.2734x; 1.2734x over previous
import jax
import jax.numpy as jnp
from jax import lax
from jax.experimental import pallas as pl
from jax.experimental.pallas import tpu as pltpu
from jax.experimental.pallas import tpu_sc as plsc

_N = 10000
_E = 320000
_D = 128

_NC = 2
_NS = 16
_NW = _NC * _NS
_CHUNK = 128
_BCH = 16
_B0 = 7
_B1 = 3
_CHMAX = _BCH * max(_B0, _B1)
_NCH = _NS * _BCH * (_B0 + _B1)
_EPAD = _NCH * _CHUNK
_ACC = 10240
_STRIPE = _ACC // _NS
_NPAD = 10016

_MESH = plsc.VectorSubcoreMesh(
    core_axis_name="c", subcore_axis_name="s", num_cores=_NC, num_subcores=_NS)
_SC_PARAMS = pltpu.CompilerParams(needs_layout_passes=False)


def _fill16(ref, val):
  v = jnp.full((16,), val, jnp.float32)
  for g in range(8):
    ref[pl.ds(16 * g, 16)] = v


def _zero_rows(ref):
  z = jnp.zeros((16,), jnp.float32)

  def body(r, _):
    for g in range(8):
      ref[r, pl.ds(16 * g, 16)] = z
    return 0

  lax.fori_loop(0, ref.shape[0], body, 0)


def _deg_body(dst_hbm, degp_hbm, dst_sm, ones_v, zrow_v, acc_sh):
  cid = lax.axis_index("c")
  sid = lax.axis_index("s")
  wid = cid * _NS + sid

  _fill16(ones_v, 1.0)
  _fill16(zrow_v, 0.0)
  for k in range(_STRIPE // 128):
    pltpu.sync_copy(zrow_v, acc_sh.at[pl.ds(sid * _STRIPE + 128 * k, 128)])
  plsc.subcore_barrier()

  def blk(b, _):
    pltpu.sync_copy(dst_hbm.at[wid, pl.ds(b * _BCH, _BCH)], dst_sm)

    def body(j, _):
      pltpu.sync_copy(ones_v, acc_sh.at[dst_sm.at[j]], add=True)
      return 0

    lax.fori_loop(0, _BCH, body, 0)
    return 0

  lax.fori_loop(0, jnp.where(cid == 0, _B0, _B1), blk, 0)
  plsc.subcore_barrier()
  pltpu.sync_copy(acc_sh.at[pl.ds(sid * _STRIPE, _STRIPE)],
                  degp_hbm.at[cid, pl.ds(sid * _STRIPE, _STRIPE)])


def _deg_kernel(dst3):
  return pl.kernel(
      _deg_body,
      out_type=jax.ShapeDtypeStruct((_NC, _ACC), jnp.float32),
      mesh=_MESH,
      compiler_params=_SC_PARAMS,
      scratch_types=[
          pltpu.VMEM((_BCH, _CHUNK), jnp.int32),
          pltpu.VMEM((_CHUNK,), jnp.float32),
          pltpu.VMEM((_CHUNK,), jnp.float32),
          pltpu.VMEM_SHARED((_ACC,), jnp.float32),
      ],
  )(dst3)


def _nblk(cid):
  return jnp.where(cid == 0, _B0, _B1)


def _gcn_body(y_hbm, src_hbm, dst_hbm, zp_hbm,
              src_sm, dst_sm, rows0, rows1, sem0, sem1, acc_sh):
  cid = lax.axis_index("c")
  sid = lax.axis_index("s")
  wid = cid * _NS + sid

  _zero_rows(rows0)
  for k in range(_STRIPE // 128):
    pltpu.sync_copy(rows0, acc_sh.at[pl.ds(sid * _STRIPE + 128 * k, 128)])
  plsc.subcore_barrier()

  def blk(b, _):
    pltpu.sync_copy(src_hbm.at[wid, pl.ds(b * _BCH, _BCH)], src_sm)
    pltpu.sync_copy(dst_hbm.at[wid, pl.ds(b * _BCH, _BCH)], dst_sm)
    pltpu.async_copy(y_hbm.at[src_sm.at[0]], rows0, sem0)
    pltpu.async_copy(y_hbm.at[src_sm.at[1]], rows1, sem1)

    def pair(i, _):
      j0 = 2 * i
      pltpu.make_async_copy(y_hbm.at[src_sm.at[j0]], rows0, sem0).wait()
      pltpu.sync_copy(rows0, acc_sh.at[dst_sm.at[j0]], add=True)

      @pl.when(j0 + 2 < _BCH)
      def _():
        pltpu.async_copy(y_hbm.at[src_sm.at[j0 + 2]], rows0, sem0)

      pltpu.make_async_copy(y_hbm.at[src_sm.at[j0 + 1]], rows1, sem1).wait()
      pltpu.sync_copy(rows1, acc_sh.at[dst_sm.at[j0 + 1]], add=True)

      @pl.when(j0 + 3 < _BCH)
      def _():
        pltpu.async_copy(y_hbm.at[src_sm.at[j0 + 3]], rows1, sem1)

      return 0

    lax.fori_loop(0, _BCH // 2, pair, 0)
    return 0

  lax.fori_loop(0, _nblk(cid), blk, 0)
  plsc.subcore_barrier()
  pltpu.sync_copy(acc_sh.at[pl.ds(sid * _STRIPE, _STRIPE)],
                  zp_hbm.at[cid, pl.ds(sid * _STRIPE, _STRIPE)])


def _gcn_kernel(y, src3, dst3):
  return pl.kernel(
      _gcn_body,
      out_type=jax.ShapeDtypeStruct((_NC, _ACC, _D), jnp.float32),
      mesh=_MESH,
      compiler_params=_SC_PARAMS,
      scratch_types=[
          pltpu.VMEM((_BCH, _CHUNK), jnp.int32),
          pltpu.VMEM((_BCH, _CHUNK), jnp.int32),
          pltpu.VMEM((_CHUNK, _D), jnp.float32),
          pltpu.VMEM((_CHUNK, _D), jnp.float32),
          pltpu.SemaphoreType.DMA,
          pltpu.SemaphoreType.DMA,
          pltpu.VMEM_SHARED((_ACC, _D), jnp.float32),
      ],
  )(y, src3, dst3)


_GCHUNK = 64
_GBCH = 32
_GCHMAX = _GBCH * max(_B0, _B1)


def _gat_chunk(src_sm, dst_sm, gs, as_v, ad_v, w_v, j):
  for g in range(_GCHUNK // 16):
    si = src_sm[j, pl.ds(16 * g, 16)]
    di = dst_sm[j, pl.ds(16 * g, 16)]
    av = plsc.load_gather(as_v, [si])
    dv = plsc.load_gather(ad_v, [di])
    td = gs + dv
    cv = jnp.where(td > 0, td, 0.2 * td)
    t = av + dv
    t = jnp.where(t > 0, t, 0.2 * t)
    w_v[pl.ds(16 * g, 16)] = jnp.exp(t - cv)


def _gat_scale(rows_v, w_v):
  def sbody(e, _):
    we = plsc.load_gather(w_v, [jnp.full((16,), e, jnp.int32)])
    for g in range(8):
      rows_v[e, pl.ds(16 * g, 16)] = rows_v[e, pl.ds(16 * g, 16)] * we
    return 0

  lax.fori_loop(0, _GCHUNK, sbody, 0, unroll=2)


def _gat_body(xl_hbm, src_hbm, dst_hbm, gs_hbm, as_hbm, ad_hbm,
              nump_hbm, denp_hbm,
              src_sm, dst_sm, rows0, rows1, w0_v, w1_v, gs_v, as_v, ad_v,
              sem0, sem1, num_sh, den_sh):
  cid = lax.axis_index("c")
  sid = lax.axis_index("s")
  wid = cid * _NS + sid

  _zero_rows(rows0)
  for k in range(_STRIPE // _GCHUNK):
    pltpu.sync_copy(rows0, num_sh.at[pl.ds(sid * _STRIPE + _GCHUNK * k, _GCHUNK)])
  for k in range(_STRIPE // 128):
    pltpu.sync_copy(rows0.at[0], den_sh.at[pl.ds(sid * _STRIPE + 128 * k, 128)])
  plsc.subcore_barrier()

  pltpu.sync_copy(gs_hbm, gs_v)
  pltpu.sync_copy(as_hbm, as_v)
  pltpu.sync_copy(ad_hbm, ad_v)

  def blk(b, _):
    pltpu.sync_copy(src_hbm.at[wid, pl.ds(b * _GBCH, _GBCH)], src_sm)
    pltpu.sync_copy(dst_hbm.at[wid, pl.ds(b * _GBCH, _GBCH)], dst_sm)
    pltpu.async_copy(xl_hbm.at[src_sm.at[0]], rows0, sem0)
    pltpu.async_copy(xl_hbm.at[src_sm.at[1]], rows1, sem1)
    gs = gs_v[...]

    def pair(i, _):
      j0 = 2 * i
      _gat_chunk(src_sm, dst_sm, gs, as_v, ad_v, w0_v, j0)
      pltpu.make_async_copy(xl_hbm.at[src_sm.at[j0]], rows0, sem0).wait()
      _gat_scale(rows0, w0_v)
      pltpu.sync_copy(rows0, num_sh.at[dst_sm.at[j0]], add=True)
      pltpu.sync_copy(w0_v, den_sh.at[dst_sm.at[j0]], add=True)

      @pl.when(j0 + 2 < _GBCH)
      def _():
        pltpu.async_copy(xl_hbm.at[src_sm.at[j0 + 2]], rows0, sem0)

      _gat_chunk(src_sm, dst_sm, gs, as_v, ad_v, w1_v, j0 + 1)
      pltpu.make_async_copy(xl_hbm.at[src_sm.at[j0 + 1]], rows1, sem1).wait()
      _gat_scale(rows1, w1_v)
      pltpu.sync_copy(rows1, num_sh.at[dst_sm.at[j0 + 1]], add=True)
      pltpu.sync_copy(w1_v, den_sh.at[dst_sm.at[j0 + 1]], add=True)

      @pl.when(j0 + 3 < _GBCH)
      def _():
        pltpu.async_copy(xl_hbm.at[src_sm.at[j0 + 3]], rows1, sem1)

      return 0

    lax.fori_loop(0, _GBCH // 2, pair, 0)
    return 0

  lax.fori_loop(0, _nblk(cid), blk, 0)
  plsc.subcore_barrier()
  pltpu.sync_copy(num_sh.at[pl.ds(sid * _STRIPE, _STRIPE)],
                  nump_hbm.at[cid, pl.ds(sid * _STRIPE, _STRIPE)])
  pltpu.sync_copy(den_sh.at[pl.ds(sid * _STRIPE, _STRIPE)],
                  denp_hbm.at[cid, pl.ds(sid * _STRIPE, _STRIPE)])


def _gat_kernel(xl, src3g, dst3g, gsl, asl, adl):
  return pl.kernel(
      _gat_body,
      out_type=(jax.ShapeDtypeStruct((_NC, _ACC, _D), jnp.float32),
                jax.ShapeDtypeStruct((_NC, _ACC), jnp.float32)),
      mesh=_MESH,
      compiler_params=_SC_PARAMS,
      scratch_types=[
          pltpu.VMEM((_GBCH, _GCHUNK), jnp.int32),
          pltpu.VMEM((_GBCH, _GCHUNK), jnp.int32),
          pltpu.VMEM((_GCHUNK, _D), jnp.float32),
          pltpu.VMEM((_GCHUNK, _D), jnp.float32),
          pltpu.VMEM((_GCHUNK,), jnp.float32),
          pltpu.VMEM((_GCHUNK,), jnp.float32),
          pltpu.VMEM((16,), jnp.float32),
          pltpu.VMEM((_NPAD,), jnp.float32),
          pltpu.VMEM((_NPAD,), jnp.float32),
          pltpu.SemaphoreType.DMA,
          pltpu.SemaphoreType.DMA,
          pltpu.VMEM_SHARED((_ACC, _D), jnp.float32),
          pltpu.VMEM_SHARED((_ACC,), jnp.float32),
      ],
  )(xl, src3g, dst3g, gsl, asl, adl)


def _dot(a, b):
  return jnp.dot(a, b, preferred_element_type=jnp.float32)


def _tc_a_body(ev, w1, b1, w2, b2, wg, degs, y_ref, dis_ref):
  h = jnp.maximum(_dot(ev[...], w1[...]) + b1[...], 0.0)
  emb = _dot(h, w2[...]) + b2[...]
  xt = _dot(emb, wg[...])
  deg = jnp.sum(degs[...], axis=1, keepdims=True) + 1.0
  dis = lax.rsqrt(deg)
  y_ref[...] = xt * dis
  dis_ref[...] = dis


def _tc_a(ev, w1, b1, w2, b2, wg, degs):
  return pl.pallas_call(
      _tc_a_body,
      out_shape=(jax.ShapeDtypeStruct((_N, _D), jnp.float32),
                 jax.ShapeDtypeStruct((_N, 1), jnp.float32)),
  )(ev, w1, b1, w2, b2, wg, degs)


def _tc_b_body(y, dis, z0, z1, wgat, asrc, adst, bgcn, xl_ref, sc_ref):
  zsum = z0[...] + z1[...] + y[...]
  gcn = jnp.maximum(dis[...] * zsum + bgcn[...], 0.0)
  xl = _dot(gcn, wgat[...])
  a_s = _dot(xl, asrc[...])
  a_d = _dot(xl, adst[...])
  gmax = jnp.max(a_s)
  td = gmax + a_d
  cc = jnp.where(td > 0, td, 0.2 * td)
  ts = a_s + a_d
  ts = jnp.where(ts > 0, ts, 0.2 * ts)
  wself = jnp.exp(ts - cc)
  gbc = jnp.full((_N, 1), 1.0, jnp.float32) * gmax
  xl_ref[...] = xl
  sc_ref[...] = jnp.concatenate([a_s, a_d, wself, gbc], axis=1)


def _tc_b(y, dis, z0, z1, wgat, asrc, adst, bgcn):
  return pl.pallas_call(
      _tc_b_body,
      out_shape=(jax.ShapeDtypeStruct((_N, _D), jnp.float32),
                 jax.ShapeDtypeStruct((_N, 4), jnp.float32)),
  )(y, dis, z0, z1, wgat, asrc, adst, bgcn)


_BLK = 2000


def _tc_c_body(h, xl, n0, n1, dens, sc, bgat, wg, bg, wr1, br1, wr2, br2,
               ws1, bs1, ws2, bs2, d_ref, hf_ref, p_ref):
  H = h[...]
  XL = xl[...]
  ws = sc[...][:, 2:3]
  den = jnp.sum(dens[...], axis=1, keepdims=True) + ws
  num = n0[...] + n1[...] + ws * XL
  diff = jnp.maximum(num / jnp.maximum(den, 1e-16) + bgat[...], 0.0)
  WG = wg[...]
  gate = jax.nn.sigmoid(_dot(H, WG[:128]) + _dot(diff, WG[128:]) + bg[...])
  WR1 = wr1[...]
  hr = jnp.maximum(_dot(H, WR1[:128]) + _dot(diff, WR1[128:]) + br1[...], 0.0)
  draw = _dot(hr, wr2[...]) + br2[...]
  delta = gate * draw
  hf = H + delta
  p = _dot(jnp.maximum(_dot(hf, ws1[...]) + bs1[...], 0.0), ws2[...]) + bs2[...]
  d_ref[...] = delta
  hf_ref[...] = hf
  p_ref[...] = p


def _tc_c(h, xl, n0, n1, dens, sc, bgat, wg, bg, wr1, br1, wr2, br2,
          ws1, bs1, ws2, bs2):
  nb = _N // _BLK

  def full(shape):
    return pl.BlockSpec(shape, lambda i: tuple(0 for _ in shape))

  return pl.pallas_call(
      _tc_c_body,
      grid=(nb,),
      in_specs=[
          pl.BlockSpec((_BLK, _D), lambda i: (i, 0)),
          pl.BlockSpec((_BLK, _D), lambda i: (i, 0)),
          pl.BlockSpec((_BLK, _D), lambda i: (i, 0)),
          pl.BlockSpec((_BLK, _D), lambda i: (i, 0)),
          pl.BlockSpec((_BLK, 2), lambda i: (i, 0)),
          pl.BlockSpec((_BLK, 4), lambda i: (i, 0)),
          full((1, _D)),
          full((2 * _D, _D)),
          full((1, _D)),
          full((2 * _D, _D)),
          full((1, _D)),
          full((_D, _D)),
          full((1, _D)),
          full((_D, 32)),
          full((1, 32)),
          full((32, 1)),
          full((1, 1)),
      ],
      out_specs=[
          pl.BlockSpec((_BLK, _D), lambda i: (i, 0)),
          pl.BlockSpec((_BLK, _D), lambda i: (i, 0)),
          pl.BlockSpec((_BLK, 1), lambda i: (i, 0)),
      ],
      out_shape=(jax.ShapeDtypeStruct((_N, _D), jnp.float32),
                 jax.ShapeDtypeStruct((_N, _D), jnp.float32),
                 jax.ShapeDtypeStruct((_N, 1), jnp.float32)),
  )(h, xl, n0, n1, dens, sc, bgat, wg, bg, wr1, br1, wr2, br2,
    ws1, bs1, ws2, bs2)


@jax.jit
def kernel(H_adapted_t, event_vector, edge_index,
           W_e1, b_e1, W_e2, b_e2,
           W_gcn, b_gcn,
           W_gat, a_src, a_dst, b_gat,
           W_gate, b_gate,
           W_r1, b_r1, W_r2, b_r2,
           W_s1, b_s1, W_s2, b_s2):
  src = edge_index[0]
  dst = edge_index[1]
  pad = _EPAD - _E
  i32 = jnp.int32
  src_p = jnp.concatenate([src, jnp.zeros((pad,), i32)])
  dst_p = jnp.concatenate([dst, jnp.full((pad,), _N, i32)])

  def skew(flat, nchunk, chunk, bch):
    f = flat.reshape(nchunk, chunk)
    n0 = _NS * bch * _B0
    chmax = bch * max(_B0, _B1)
    a = f[:n0].reshape(_NS, bch * _B0, chunk)
    a = jnp.pad(a, ((0, 0), (0, chmax - bch * _B0), (0, 0)))
    b = f[n0:].reshape(_NS, bch * _B1, chunk)
    b = jnp.pad(b, ((0, 0), (0, chmax - bch * _B1), (0, 0)))
    return jnp.concatenate([a, b], axis=0)

  src3 = skew(src_p, _NCH, _CHUNK, _BCH)
  dst3 = skew(dst_p, _NCH, _CHUNK, _BCH)

  degp = _deg_kernel(dst3)
  degs = jnp.stack([degp[0, :_N], degp[1, :_N]], axis=1)

  y, dis = _tc_a(event_vector, W_e1, b_e1.reshape(1, _D), W_e2,
                 b_e2.reshape(1, _D), W_gcn, degs)

  zp = _gcn_kernel(y, src3, dst3)

  xl, sc = _tc_b(y, dis, zp[0, :_N], zp[1, :_N], W_gat,
                 a_src.reshape(_D, 1), a_dst.reshape(_D, 1),
                 b_gcn.reshape(1, _D))

  zpadf = jnp.zeros((_NPAD - _N,), jnp.float32)
  asl = jnp.concatenate([sc[:, 0], zpadf])
  adl = jnp.concatenate([sc[:, 1], zpadf])
  gsl = sc[0:16, 3]

  src3g = skew(src_p, 2 * _NCH, _GCHUNK, _GBCH)
  dst3g = skew(dst_p, 2 * _NCH, _GCHUNK, _GBCH)
  nump, denp = _gat_kernel(xl, src3g, dst3g, gsl, asl, adl)
  dens = jnp.stack([denp[0, :_N], denp[1, :_N]], axis=1)

  delta, hf, p = _tc_c(H_adapted_t, xl, nump[0, :_N], nump[1, :_N], dens, sc,
                       b_gat.reshape(1, _D), W_gate, b_gate.reshape(1, _D),
                       W_r1, b_r1.reshape(1, _D), W_r2, b_r2.reshape(1, _D),
                       W_s1, b_s1.reshape(1, 32), W_s2, b_s2.reshape(1, 1))
  return delta, hf, p.reshape(_N)
